# initial kernel scaffold (unmeasured)
import jax
import jax.numpy as jnp
from jax import lax
from jax.experimental import pallas as pl
from jax.experimental.pallas import tpu as pltpu

N_DEV = 4
B = 16
H = 16
D = 64
HD = H * D
PACK = HD + 2 * H
SCALE = D ** -0.5


def _body(q_ref, k_ref, v_ref, out_ref, part_ref, comm_ref, send_sems, recv_sems):
    b = pl.program_id(0)
    my = lax.axis_index("i")
    kv = k_ref.shape[0]

    c_ids = lax.broadcasted_iota(jnp.int32, (H, HD), 1)
    h_ids = lax.broadcasted_iota(jnp.int32, (H, HD), 0)
    maskT = (c_ids // D) == h_ids
    maskT_bf = maskT.astype(jnp.bfloat16)

    q = q_ref[:, :]
    aqT = jnp.where(maskT, jnp.broadcast_to(q, (H, HD)), 0.0).astype(jnp.bfloat16)

    kmat = k_ref[:, :].astype(jnp.bfloat16)
    s = lax.dot_general(
        kmat, aqT, (((1,), (1,)), ((), ())),
        preferred_element_type=jnp.float32,
    ) * SCALE
    m = jnp.max(s, axis=0, keepdims=True)
    p = jnp.exp(s - m)
    l = jnp.sum(p, axis=0, keepdims=True)

    p2 = lax.dot_general(
        p.astype(jnp.bfloat16), maskT_bf, (((1,), (0,)), ((), ())),
        preferred_element_type=jnp.bfloat16,
    )
    x = p2 * v_ref[:, :].astype(jnp.bfloat16)
    ones = jnp.ones((1, kv), jnp.bfloat16)
    o = lax.dot_general(
        ones, x, (((1,), (0,)), ((), ())),
        preferred_element_type=jnp.float32,
    )

    part_ref[pl.ds(b, 1), 0:HD] = o
    part_ref[pl.ds(b, 1), HD:HD + H] = m
    part_ref[pl.ds(b, 1), HD + H:PACK] = l

    @pl.when(b == B - 1)
    def _():
        comm_ref[pl.ds(my, 1), :, :] = part_ref[:, :][None, :, :]

        sends = []
        for t in range(1, N_DEV):
            peer = lax.rem(my + t, N_DEV)
            rdma = pltpu.make_async_remote_copy(
                src_ref=part_ref,
                dst_ref=comm_ref.at[my],
                send_sem=send_sems.at[t - 1],
                recv_sem=recv_sems.at[t - 1],
                device_id=(peer,),
                device_id_type=pl.DeviceIdType.MESH,
            )
            rdma.start()
            sends.append(rdma)

        for t in range(1, N_DEV):
            src_dev = lax.rem(my - t + N_DEV, N_DEV)
            recv = pltpu.make_async_remote_copy(
                src_ref=part_ref,
                dst_ref=comm_ref.at[src_dev],
                send_sem=send_sems.at[t - 1],
                recv_sem=recv_sems.at[t - 1],
                device_id=(src_dev,),
                device_id_type=pl.DeviceIdType.MESH,
            )
            recv.wait_recv()

        for rdma in sends:
            rdma.wait_send()

        ms = [comm_ref[i, :, HD:HD + H] for i in range(N_DEV)]
        mg = jnp.maximum(
            jnp.maximum(ms[0], ms[1]), jnp.maximum(ms[2], ms[3])
        )
        num = jnp.zeros((B, HD), jnp.float32)
        den = jnp.zeros((B, H), jnp.float32)
        for i in range(N_DEV):
            w = jnp.exp(ms[i] - mg)
            den = den + comm_ref[i, :, HD + H:PACK] * w
            w2 = lax.dot_general(
                w.astype(jnp.bfloat16), maskT_bf, (((1,), (0,)), ((), ())),
                preferred_element_type=jnp.float32,
            )
            num = num + comm_ref[i, :, 0:HD] * w2
        den2 = lax.dot_general(
            den.astype(jnp.bfloat16), maskT_bf, (((1,), (0,)), ((), ())),
            preferred_element_type=jnp.float32,
        )
        out_ref[:, :] = num / den2


def kernel(Q, K, V):
    kv = K.shape[1]
    Qr = Q.reshape(B, HD)
    Kr = K.reshape(B, kv, HD)
    Vr = V.reshape(B, kv, HD)

    out = pl.pallas_call(
        _body,
        grid=(B,),
        in_specs=[
            pl.BlockSpec((1, HD), lambda b: (b, 0)),
            pl.BlockSpec((None, kv, HD), lambda b: (b, 0, 0)),
            pl.BlockSpec((None, kv, HD), lambda b: (b, 0, 0)),
        ],
        out_specs=pl.BlockSpec((B, HD), lambda b: (0, 0)),
        out_shape=jax.ShapeDtypeStruct((B, HD), jnp.float32),
        scratch_shapes=[
            pltpu.VMEM((B, PACK), jnp.float32),
            pltpu.VMEM((N_DEV, B, PACK), jnp.float32),
            pltpu.SemaphoreType.DMA((N_DEV - 1,)),
            pltpu.SemaphoreType.DMA((N_DEV - 1,)),
        ],
        compiler_params=pltpu.CompilerParams(collective_id=0),
    )(Qr, Kr, Vr)
    return out.reshape(B, 1, H, D)


# baseline (device time: 191179 ns/iter reference)
import jax
import jax.numpy as jnp
from jax import lax
from jax.experimental import pallas as pl
from jax.experimental.pallas import tpu as pltpu

N_DEV = 4
B = 16
H = 16
D = 64
HD = H * D
PACK = HD + 256
SCALE = D ** -0.5


def _body(q_ref, k_ref, v_ref, out_ref, part_ref, comm_ref, send_sems, recv_sems):
    b = pl.program_id(0)
    my = lax.axis_index("i")
    kv = k_ref.shape[0]

    c_ids = lax.broadcasted_iota(jnp.int32, (H, HD), 1)
    h_ids = lax.broadcasted_iota(jnp.int32, (H, HD), 0)
    maskT = (c_ids // D) == h_ids
    maskT_bf = maskT.astype(jnp.bfloat16)

    q = q_ref[pl.ds(b, 1), :]
    aqT = jnp.where(maskT, jnp.broadcast_to(q, (H, HD)), 0.0).astype(jnp.bfloat16)

    kmat = k_ref[:, :].astype(jnp.bfloat16)
    s = lax.dot_general(
        kmat, aqT, (((1,), (1,)), ((), ())),
        preferred_element_type=jnp.float32,
    ) * SCALE
    m = jnp.max(s, axis=0, keepdims=True)
    p = jnp.exp(s - m)
    l = jnp.sum(p, axis=0, keepdims=True)

    p2 = lax.dot_general(
        p.astype(jnp.bfloat16), maskT_bf, (((1,), (0,)), ((), ())),
        preferred_element_type=jnp.float32,
    ).astype(jnp.bfloat16)
    x = p2 * v_ref[:, :].astype(jnp.bfloat16)
    ones = jnp.ones((1, kv), jnp.bfloat16)
    o = lax.dot_general(
        ones, x, (((1,), (0,)), ((), ())),
        preferred_element_type=jnp.float32,
    )

    row = jnp.concatenate(
        [o, m, l, jnp.zeros((1, PACK - HD - 2 * H), jnp.float32)], axis=1
    )
    part_ref[pl.ds(b, 1), :] = row

    @pl.when(b == B - 1)
    def _():
        comm_ref[pl.ds(my, 1), :, :] = part_ref[:, :][None, :, :]

        sends = []
        for t in range(1, N_DEV):
            peer = lax.rem(my + t, N_DEV)
            rdma = pltpu.make_async_remote_copy(
                src_ref=part_ref,
                dst_ref=comm_ref.at[my],
                send_sem=send_sems.at[t - 1],
                recv_sem=recv_sems.at[t - 1],
                device_id=(peer,),
                device_id_type=pl.DeviceIdType.MESH,
            )
            rdma.start()
            sends.append(rdma)

        for t in range(1, N_DEV):
            src_dev = lax.rem(my - t + N_DEV, N_DEV)
            recv = pltpu.make_async_remote_copy(
                src_ref=part_ref,
                dst_ref=comm_ref.at[src_dev],
                send_sem=send_sems.at[t - 1],
                recv_sem=recv_sems.at[t - 1],
                device_id=(src_dev,),
                device_id_type=pl.DeviceIdType.MESH,
            )
            recv.wait_recv()

        for rdma in sends:
            rdma.wait_send()

        ms = [comm_ref[i, :, HD:HD + H] for i in range(N_DEV)]
        mg = jnp.maximum(
            jnp.maximum(ms[0], ms[1]), jnp.maximum(ms[2], ms[3])
        )
        num = jnp.zeros((B, HD), jnp.float32)
        den = jnp.zeros((B, H), jnp.float32)
        for i in range(N_DEV):
            w = jnp.exp(ms[i] - mg)
            den = den + comm_ref[i, :, HD + H:HD + 2 * H] * w
            w2 = lax.dot_general(
                w.astype(jnp.bfloat16), maskT_bf, (((1,), (0,)), ((), ())),
                preferred_element_type=jnp.float32,
            )
            num = num + comm_ref[i, :, 0:HD] * w2
        den2 = lax.dot_general(
            den.astype(jnp.bfloat16), maskT_bf, (((1,), (0,)), ((), ())),
            preferred_element_type=jnp.float32,
        )
        out_ref[:, :] = num / den2


def kernel(Q, K, V):
    kv = K.shape[1]
    Qr = Q.reshape(B, HD)
    Kr = K.reshape(B, kv, HD)
    Vr = V.reshape(B, kv, HD)

    out = pl.pallas_call(
        _body,
        grid=(B,),
        in_specs=[
            pl.BlockSpec((B, HD), lambda b: (0, 0)),
            pl.BlockSpec((None, kv, HD), lambda b: (b, 0, 0)),
            pl.BlockSpec((None, kv, HD), lambda b: (b, 0, 0)),
        ],
        out_specs=pl.BlockSpec((B, HD), lambda b: (0, 0)),
        out_shape=jax.ShapeDtypeStruct((B, HD), jnp.float32),
        scratch_shapes=[
            pltpu.VMEM((B, PACK), jnp.float32),
            pltpu.VMEM((N_DEV, B, PACK), jnp.float32),
            pltpu.SemaphoreType.DMA((N_DEV - 1,)),
            pltpu.SemaphoreType.DMA((N_DEV - 1,)),
        ],
    )(Qr, Kr, Vr)
    return out.reshape(B, 1, H, D)


# device time: 184600 ns/iter; 1.0356x vs baseline; 1.0356x over previous
import jax
import jax.numpy as jnp
from jax import lax
from jax.experimental import pallas as pl
from jax.experimental.pallas import tpu as pltpu

N_DEV = 4
B = 16
H = 16
D = 64
HD = H * D
PACK = HD + 256
SCALE = D ** -0.5


def _body(q_ref, k_ref, v_ref, out_ref, part_ref, comm_ref, send_sems, recv_sems):
    b = pl.program_id(0)
    my = lax.axis_index("i")
    kv = k_ref.shape[0]

    c_ids = lax.broadcasted_iota(jnp.int32, (H, HD), 1)
    h_ids = lax.broadcasted_iota(jnp.int32, (H, HD), 0)
    maskT = (c_ids // D) == h_ids
    maskT_bf = maskT.astype(jnp.bfloat16)

    q = q_ref[pl.ds(b, 1), :]
    qT = jnp.transpose(q)
    c_ids2 = lax.broadcasted_iota(jnp.int32, (HD, H), 0)
    h_ids2 = lax.broadcasted_iota(jnp.int32, (HD, H), 1)
    aq = jnp.where(
        (c_ids2 // D) == h_ids2, jnp.broadcast_to(qT, (HD, H)), 0.0
    )

    s = lax.dot_general(
        k_ref[:, :], aq, (((1,), (0,)), ((), ())),
        preferred_element_type=jnp.float32,
    ) * SCALE
    m = jnp.max(s, axis=0, keepdims=True)
    p = jnp.exp(s - m)
    l = jnp.sum(p, axis=0, keepdims=True)

    pT = jnp.transpose(p)
    o_full = lax.dot_general(
        pT, v_ref[:, :], (((1,), (0,)), ((), ())),
        preferred_element_type=jnp.float32,
    )
    o = jnp.sum(
        jnp.where(maskT, o_full, 0.0), axis=0, keepdims=True
    )

    row = jnp.concatenate(
        [o, m, l, jnp.zeros((1, PACK - HD - 2 * H), jnp.float32)], axis=1
    )
    part_ref[pl.ds(b, 1), :] = row

    @pl.when(b == B - 1)
    def _():
        comm_ref[pl.ds(my, 1), :, :] = part_ref[:, :][None, :, :]

        sends = []
        for t in range(1, N_DEV):
            peer = lax.rem(my + t, N_DEV)
            rdma = pltpu.make_async_remote_copy(
                src_ref=part_ref,
                dst_ref=comm_ref.at[my],
                send_sem=send_sems.at[t - 1],
                recv_sem=recv_sems.at[t - 1],
                device_id=(peer,),
                device_id_type=pl.DeviceIdType.MESH,
            )
            rdma.start()
            sends.append(rdma)

        for t in range(1, N_DEV):
            src_dev = lax.rem(my - t + N_DEV, N_DEV)
            recv = pltpu.make_async_remote_copy(
                src_ref=part_ref,
                dst_ref=comm_ref.at[src_dev],
                send_sem=send_sems.at[t - 1],
                recv_sem=recv_sems.at[t - 1],
                device_id=(src_dev,),
                device_id_type=pl.DeviceIdType.MESH,
            )
            recv.wait_recv()

        for rdma in sends:
            rdma.wait_send()

        ms = [comm_ref[i, :, HD:HD + H] for i in range(N_DEV)]
        mg = jnp.maximum(
            jnp.maximum(ms[0], ms[1]), jnp.maximum(ms[2], ms[3])
        )
        num = jnp.zeros((B, HD), jnp.float32)
        den = jnp.zeros((B, H), jnp.float32)
        for i in range(N_DEV):
            w = jnp.exp(ms[i] - mg)
            den = den + comm_ref[i, :, HD + H:HD + 2 * H] * w
            w2 = lax.dot_general(
                w.astype(jnp.bfloat16), maskT_bf, (((1,), (0,)), ((), ())),
                preferred_element_type=jnp.float32,
            )
            num = num + comm_ref[i, :, 0:HD] * w2
        den2 = lax.dot_general(
            den.astype(jnp.bfloat16), maskT_bf, (((1,), (0,)), ((), ())),
            preferred_element_type=jnp.float32,
        )
        out_ref[:, :] = num / den2


def kernel(Q, K, V):
    kv = K.shape[1]
    Qr = Q.reshape(B, HD)
    Kr = K.reshape(B, kv, HD)
    Vr = V.reshape(B, kv, HD)

    out = pl.pallas_call(
        _body,
        grid=(B,),
        in_specs=[
            pl.BlockSpec((B, HD), lambda b: (0, 0)),
            pl.BlockSpec((None, kv, HD), lambda b: (b, 0, 0)),
            pl.BlockSpec((None, kv, HD), lambda b: (b, 0, 0)),
        ],
        out_specs=pl.BlockSpec((B, HD), lambda b: (0, 0)),
        out_shape=jax.ShapeDtypeStruct((B, HD), jnp.float32),
        scratch_shapes=[
            pltpu.VMEM((B, PACK), jnp.float32),
            pltpu.VMEM((N_DEV, B, PACK), jnp.float32),
            pltpu.SemaphoreType.DMA((N_DEV - 1,)),
            pltpu.SemaphoreType.DMA((N_DEV - 1,)),
        ],
    )(Qr, Kr, Vr)
    return out.reshape(B, 1, H, D)
